# Initial kernel scaffold; baseline (speedup 1.0000x reference)
import jax
import jax.numpy as jnp
from jax.experimental import pallas as pl


def kernel(x, edge_index, l1_W1, l1_b1, l1_W2, l1_b2, l2_W1, l2_b1, l2_W2, l2_b2):
    raise NotImplementedError("write your pallas kernel here")


# speedup vs baseline: 1.4919x; 1.4919x over previous
import dataclasses
import functools

import jax
import jax.numpy as jnp
from jax import lax
from jax.experimental import pallas as pl
from jax.experimental.pallas import tpu as pltpu
from jax.experimental.pallas import tpu_sc as plsc

NC = 2
NS = 16
NW = NC * NS
LANES = 16

_NEG_INF = float("-inf")


def _vmesh():
    return plsc.VectorSubcoreMesh(core_axis_name="c", subcore_axis_name="s")


def _sc_params():
    cp = pltpu.CompilerParams()
    if "needs_layout_passes" in pltpu.CompilerParams.__dataclass_fields__:
        cp = dataclasses.replace(cp, needs_layout_passes=False)
    return cp


def _pq_body(f, x_ref, w1_ref, b1_ref, p_ref, q_ref):
    xv = x_ref[...]
    wa = w1_ref[:f, :]
    wb = w1_ref[f:, :]
    qv = jnp.dot(xv, wb, preferred_element_type=jnp.float32)
    p_ref[...] = jnp.dot(xv, wa, preferred_element_type=jnp.float32) - qv \
        + b1_ref[...]
    q_ref[...] = qv


def _stage_a(x, w1, b1, n, f, c):
    return pl.pallas_call(
        functools.partial(_pq_body, f),
        out_shape=(jax.ShapeDtypeStruct((n, c), jnp.float32),
                   jax.ShapeDtypeStruct((n, c), jnp.float32)),
    )(x, w1, b1.reshape(1, c))


def _stage_b(p, q, dst, src, e, c):
    epw = e // NW
    ecb = 80
    assert epw % ecb == 0

    @functools.partial(
        pl.kernel,
        out_type=(jax.ShapeDtypeStruct((e, c), jnp.float32),
                  jax.ShapeDtypeStruct((e, c), jnp.float32)),
        mesh=_vmesh(),
        compiler_params=_sc_params(),
        scratch_types=[
            pltpu.VMEM((ecb,), jnp.int32),
            pltpu.VMEM((ecb,), jnp.int32),
            pltpu.VMEM((ecb, c), jnp.float32),
            pltpu.VMEM((ecb, c), jnp.float32),
            pltpu.SemaphoreType.DMA,
            pltpu.SemaphoreType.DMA,
        ],
    )
    def k(p_hbm, q_hbm, dst_hbm, src_hbm, pg_hbm, qg_hbm,
          didx, sidx, pbuf, qbuf, sem1, sem2):
        w = lax.axis_index("s") * NC + lax.axis_index("c")
        base = w * epw

        @pl.loop(0, epw, step=ecb)
        def _(off):
            pltpu.sync_copy(dst_hbm.at[pl.ds(base + off, ecb)], didx)
            pltpu.sync_copy(src_hbm.at[pl.ds(base + off, ecb)], sidx)
            c1 = pltpu.async_copy(p_hbm.at[didx], pbuf, sem1)
            c2 = pltpu.async_copy(q_hbm.at[sidx], qbuf, sem2)
            c1.wait()
            c2.wait()
            pltpu.sync_copy(pbuf, pg_hbm.at[pl.ds(base + off, ecb)])
            pltpu.sync_copy(qbuf, qg_hbm.at[pl.ds(base + off, ecb)])

    return k(p, q, dst, src)


def _ht_body(pg_ref, qg_ref, w2_ref, b2_ref, ht_ref):
    m = jnp.maximum(pg_ref[...] + qg_ref[...], 0.0)
    ht_ref[...] = lax.dot_general(
        w2_ref[...], m, (((0,), (1,)), ((), ())),
        preferred_element_type=jnp.float32) + b2_ref[...]


def _stage_c(pg, qg, w2, b2, e, c):
    be = 2560
    assert e % be == 0
    return pl.pallas_call(
        _ht_body,
        grid=(e // be,),
        in_specs=[
            pl.BlockSpec((be, c), lambda i: (i, 0)),
            pl.BlockSpec((be, c), lambda i: (i, 0)),
            pl.BlockSpec((c, c), lambda i: (0, 0)),
            pl.BlockSpec((c, 1), lambda i: (0, 0)),
        ],
        out_specs=pl.BlockSpec((c, be), lambda i: (0, i)),
        out_shape=jax.ShapeDtypeStruct((c, e), jnp.float32),
    )(pg, qg, w2, b2.reshape(c, 1))


def _scatter_max1(acc, idx, vals):
    cur = plsc.load_gather(acc, [idx])
    new = jnp.maximum(cur, vals)
    plsc.store_scatter(acc, [idx], new)
    chk = plsc.load_gather(acc, [idx])
    nbad = jnp.sum((chk < new).astype(jnp.int32))

    def cond(cs):
        return cs[0] > 0

    def body(cs):
        _, want = cs
        c2 = plsc.load_gather(acc, [idx])
        plsc.store_scatter(acc, [idx], jnp.maximum(c2, want), mask=c2 < want)
        c3 = plsc.load_gather(acc, [idx])
        return jnp.sum((c3 < want).astype(jnp.int32)), want

    lax.while_loop(cond, body, (nbad, new))


def _stage_d(ht1d, dst, n, e, c):
    rpw = c // NW
    ecd = 4000
    assert e % ecd == 0

    @functools.partial(
        pl.kernel,
        out_type=jax.ShapeDtypeStruct((c * n,), jnp.float32),
        mesh=_vmesh(),
        compiler_params=_sc_params(),
        scratch_types=[
            pltpu.VMEM((ecd,), jnp.int32),
            pltpu.VMEM((rpw * ecd,), jnp.float32),
            pltpu.VMEM((rpw * n,), jnp.float32),
        ],
    )
    def k(ht_hbm, dst_hbm, out_hbm, dstb, hb, acc):
        w = lax.axis_index("s") * NC + lax.axis_index("c")
        rbase = w * rpw

        @pl.loop(0, rpw * n, step=LANES)
        def _(i):
            acc[pl.ds(i, LANES)] = jnp.zeros((LANES,), jnp.float32)

        @pl.loop(0, e, step=ecd)
        def _(off):
            pltpu.sync_copy(dst_hbm.at[pl.ds(off, ecd)], dstb)
            for r in range(rpw):
                pltpu.sync_copy(
                    ht_hbm.at[pl.ds((rbase + r) * e + off, ecd)],
                    hb.at[pl.ds(r * ecd, ecd)])

            @pl.loop(0, ecd, step=LANES)
            def _(i):
                idx = dstb[pl.ds(i, LANES)]
                for r in range(rpw):
                    vals = hb[pl.ds(r * ecd + i, LANES)]
                    idxr = idx + jnp.full((LANES,), r * n, jnp.int32)
                    _scatter_max1(acc, idxr, vals)

        pltpu.sync_copy(acc, out_hbm.at[pl.ds(rbase * n, rpw * n)])

    return k(ht1d, dst)


def _pq2_body(c, ht_ref, w_ref, b_ref, p2_ref, q2_ref):
    h = ht_ref[...]
    wa = w_ref[:c, :]
    wb = w_ref[c:, :]
    q2 = lax.dot_general(h, wb, (((0,), (0,)), ((), ())),
                         preferred_element_type=jnp.float32)
    p2 = lax.dot_general(h, wa, (((0,), (0,)), ((), ())),
                         preferred_element_type=jnp.float32) - q2 + b_ref[...]
    p2_ref[...] = p2
    q2_ref[...] = q2


def _stage_e(ht, w, b, n, c, out):
    return pl.pallas_call(
        functools.partial(_pq2_body, c),
        out_shape=(jax.ShapeDtypeStruct((n, out), jnp.float32),
                   jax.ShapeDtypeStruct((n, out), jnp.float32)),
    )(ht, w, b.reshape(1, out))


def _stage_f(p2f, q2f, dst, src, wv, n, e):
    epw = e // NW
    ecf = 2000
    assert epw % ecf == 0
    n2 = 2 * n

    @functools.partial(
        pl.kernel,
        out_type=jax.ShapeDtypeStruct((NW * n2,), jnp.float32),
        mesh=_vmesh(),
        compiler_params=_sc_params(),
        scratch_types=[
            pltpu.VMEM((n2,), jnp.float32),
            pltpu.VMEM((n2,), jnp.float32),
            pltpu.VMEM((n2,), jnp.float32),
            pltpu.VMEM((ecf,), jnp.int32),
            pltpu.VMEM((ecf,), jnp.int32),
            pltpu.VMEM((6 * LANES,), jnp.float32),
        ],
    )
    def k(p2_hbm, q2_hbm, dst_hbm, src_hbm, wv_hbm, part_hbm,
          p2b, q2b, accf, db, sb, wb6):
        w = lax.axis_index("s") * NC + lax.axis_index("c")
        base = w * epw
        pltpu.sync_copy(p2_hbm, p2b)
        pltpu.sync_copy(q2_hbm, q2b)
        pltpu.sync_copy(wv_hbm, wb6)

        @pl.loop(0, n2, step=LANES)
        def _(i):
            accf[pl.ds(i, LANES)] = jnp.full((LANES,), _NEG_INF, jnp.float32)

        w00 = wb6[pl.ds(0, LANES)]
        w01 = wb6[pl.ds(16, LANES)]
        w10 = wb6[pl.ds(32, LANES)]
        w11 = wb6[pl.ds(48, LANES)]
        b0 = wb6[pl.ds(64, LANES)]
        b1 = wb6[pl.ds(80, LANES)]

        @pl.loop(0, epw, step=ecf)
        def _(off):
            pltpu.sync_copy(dst_hbm.at[pl.ds(base + off, ecf)], db)
            pltpu.sync_copy(src_hbm.at[pl.ds(base + off, ecf)], sb)

            @pl.loop(0, ecf, step=LANES)
            def _(i):
                d = db[pl.ds(i, LANES)]
                s = sb[pl.ds(i, LANES)]
                one = jnp.ones((LANES,), jnp.int32)
                d2 = d + d
                s2 = s + s
                p0 = plsc.load_gather(p2b, [d2])
                p1 = plsc.load_gather(p2b, [d2 + one])
                q0 = plsc.load_gather(q2b, [s2])
                q1 = plsc.load_gather(q2b, [s2 + one])
                g0 = jnp.maximum(p0 + q0, 0.0)
                g1 = jnp.maximum(p1 + q1, 0.0)
                o0 = g0 * w00 + g1 * w10 + b0
                o1 = g0 * w01 + g1 * w11 + b1
                _scatter_max1(accf, d2, o0)
                _scatter_max1(accf, d2 + one, o1)

        pltpu.sync_copy(accf, part_hbm.at[pl.ds(w * n2, n2)])

    return k(p2f, q2f, dst, src, wv)


def _combine_body(p_ref, o_ref):
    m = jnp.max(p_ref[...], axis=0, keepdims=True)
    o_ref[...] = jnp.where(m == _NEG_INF, 0.0, m)


def _stage_g(parts, n):
    n2 = 2 * n
    return pl.pallas_call(
        _combine_body,
        out_shape=jax.ShapeDtypeStruct((1, n2), jnp.float32),
    )(parts)


def kernel(x, edge_index, l1_W1, l1_b1, l1_W2, l1_b2,
           l2_W1, l2_b1, l2_W2, l2_b2):
    n, f = x.shape
    e = edge_index.shape[1]
    c = l1_W2.shape[0]
    out = l2_W2.shape[0]

    src = edge_index[0]
    dst = edge_index[1]

    p, q = _stage_a(x, l1_W1, l1_b1, n, f, c)
    pg, qg = _stage_b(p, q, dst, src, e, c)
    ht = _stage_c(pg, qg, l1_W2, l1_b2, e, c)
    h_t = _stage_d(ht.reshape(-1), dst, n, e, c).reshape(c, n)
    p2, q2 = _stage_e(h_t, l2_W1, l2_b1, n, c, out)

    wv = jnp.concatenate(
        [jnp.broadcast_to(l2_W2.reshape(4, 1), (4, LANES)),
         jnp.broadcast_to(l2_b2.reshape(2, 1), (2, LANES))],
        axis=0).reshape(-1)
    parts = _stage_f(p2.reshape(-1), q2.reshape(-1), dst, src, wv, n, e)
    res = _stage_g(parts.reshape(NW, 2 * n), n)
    return res.reshape(n, out)


# speedup vs baseline: 3.1086x; 2.0837x over previous
import dataclasses
import functools

import jax
import jax.numpy as jnp
from jax import lax
from jax.experimental import pallas as pl
from jax.experimental.pallas import tpu as pltpu
from jax.experimental.pallas import tpu_sc as plsc

NC = 2
NS = 16
NW = NC * NS
LANES = 16

_NEG_INF = float("-inf")


def _vmesh():
    return plsc.VectorSubcoreMesh(core_axis_name="c", subcore_axis_name="s")


def _sc_params():
    cp = pltpu.CompilerParams()
    if "needs_layout_passes" in pltpu.CompilerParams.__dataclass_fields__:
        cp = dataclasses.replace(cp, needs_layout_passes=False)
    return cp


def _pq_body(f, x_ref, w1_ref, b1_ref, p_ref, q_ref):
    xv = x_ref[...]
    wa = w1_ref[:f, :]
    wb = w1_ref[f:, :]
    qv = jnp.dot(xv, wb, preferred_element_type=jnp.float32)
    p_ref[...] = jnp.dot(xv, wa, preferred_element_type=jnp.float32) - qv \
        + b1_ref[...]
    q_ref[...] = qv


def _stage_a(x, w1, b1, n, f, c):
    return pl.pallas_call(
        functools.partial(_pq_body, f),
        out_shape=(jax.ShapeDtypeStruct((n, c), jnp.float32),
                   jax.ShapeDtypeStruct((n, c), jnp.float32)),
    )(x, w1, b1.reshape(1, c))


def _stage_b(p, q, dst, src, e, c):
    epw = e // NW
    ecb = 200
    assert epw % ecb == 0

    sub = 40
    nsub = ecb // sub
    assert (epw // ecb) % 2 == 0

    @functools.partial(
        pl.kernel,
        out_type=(jax.ShapeDtypeStruct((e, c), jnp.float32),
                  jax.ShapeDtypeStruct((e, c), jnp.float32)),
        mesh=_vmesh(),
        compiler_params=_sc_params(),
        scratch_types=[
            pltpu.VMEM((epw,), jnp.int32),
            pltpu.VMEM((epw,), jnp.int32),
            pltpu.VMEM((ecb, c), jnp.float32),
            pltpu.VMEM((ecb, c), jnp.float32),
            pltpu.VMEM((ecb, c), jnp.float32),
            pltpu.VMEM((ecb, c), jnp.float32),
            pltpu.SemaphoreType.DMA,
            pltpu.SemaphoreType.DMA,
            pltpu.SemaphoreType.DMA,
        ],
    )
    def k(p_hbm, q_hbm, dst_hbm, src_hbm, pg_hbm, qg_hbm,
          dia, sia, pb0, qb0, pb1, qb1, sg, sw0, sw1):
        w = lax.axis_index("s") * NC + lax.axis_index("c")
        base = w * epw
        pltpu.sync_copy(dst_hbm.at[pl.ds(base, epw)], dia)
        pltpu.sync_copy(src_hbm.at[pl.ds(base, epw)], sia)

        def gathers(off, pb, qb):
            for j in range(nsub):
                pltpu.async_copy(
                    p_hbm.at[dia.at[pl.ds(off + j * sub, sub)]],
                    pb.at[pl.ds(j * sub, sub)], sg)
                pltpu.async_copy(
                    q_hbm.at[sia.at[pl.ds(off + j * sub, sub)]],
                    qb.at[pl.ds(j * sub, sub)], sg)

        def drain_gathers(pb, qb):
            for j in range(nsub):
                pltpu.make_async_copy(
                    p_hbm.at[dia.at[pl.ds(j * sub, sub)]],
                    pb.at[pl.ds(j * sub, sub)], sg).wait()
                pltpu.make_async_copy(
                    q_hbm.at[sia.at[pl.ds(j * sub, sub)]],
                    qb.at[pl.ds(j * sub, sub)], sg).wait()

        def write(off, pb, qb, sw):
            pltpu.async_copy(pb, pg_hbm.at[pl.ds(base + off, ecb)], sw)
            pltpu.async_copy(qb, qg_hbm.at[pl.ds(base + off, ecb)], sw)

        def wait_write(off, pb, qb, sw):
            pltpu.make_async_copy(pb, pg_hbm.at[pl.ds(base + off, ecb)],
                                  sw).wait()
            pltpu.make_async_copy(qb, qg_hbm.at[pl.ds(base + off, ecb)],
                                  sw).wait()

        gathers(0, pb0, qb0)

        @pl.loop(0, epw, step=2 * ecb)
        def _(off):
            drain_gathers(pb0, qb0)

            @pl.when(off > 0)
            def _():
                wait_write(off - ecb, pb1, qb1, sw1)

            gathers(off + ecb, pb1, qb1)
            write(off, pb0, qb0, sw0)
            drain_gathers(pb1, qb1)
            wait_write(off, pb0, qb0, sw0)

            @pl.when(off + 2 * ecb < epw)
            def _():
                gathers(off + 2 * ecb, pb0, qb0)

            write(off + ecb, pb1, qb1, sw1)

        wait_write(epw - ecb, pb1, qb1, sw1)

    return k(p, q, dst, src)


def _ht_body(pg_ref, qg_ref, w2_ref, b2_ref, ht_ref):
    m = jnp.maximum(pg_ref[...] + qg_ref[...], 0.0)
    ht_ref[...] = lax.dot_general(
        w2_ref[...], m, (((0,), (1,)), ((), ())),
        preferred_element_type=jnp.float32) + b2_ref[...]


def _stage_c(pg, qg, w2, b2, e, c):
    be = 2560
    assert e % be == 0
    return pl.pallas_call(
        _ht_body,
        grid=(e // be,),
        in_specs=[
            pl.BlockSpec((be, c), lambda i: (i, 0)),
            pl.BlockSpec((be, c), lambda i: (i, 0)),
            pl.BlockSpec((c, c), lambda i: (0, 0)),
            pl.BlockSpec((c, 1), lambda i: (0, 0)),
        ],
        out_specs=pl.BlockSpec((c, be), lambda i: (0, i)),
        out_shape=jax.ShapeDtypeStruct((c, e), jnp.float32),
    )(pg, qg, w2, b2.reshape(c, 1))


def _repair(acc, idx, want0):
    def cond(cs):
        return cs[0] > 0

    def body(cs):
        _, want = cs
        c2 = plsc.load_gather(acc, [idx])
        plsc.store_scatter(acc, [idx], jnp.maximum(c2, want), mask=c2 < want)
        c3 = plsc.load_gather(acc, [idx])
        nb = plsc.all_reduce_population_count(c3 < want)[0]
        return nb, want

    lax.while_loop(cond, body, (jnp.int32(1), want0))


def _scatter_max1(acc, idx, vals):
    cur = plsc.load_gather(acc, [idx])
    new = jnp.maximum(cur, vals)
    plsc.store_scatter(acc, [idx], new)
    chk = plsc.load_gather(acc, [idx])
    nbad = plsc.all_reduce_population_count(chk < new)[0]

    @pl.when(nbad > 0)
    def _():
        _repair(acc, idx, new)


def _stage_d(ht1d, dst, n, e, c):
    rpw = c // NW
    ecd = 4000
    assert (e // ecd) % 2 == 0

    @functools.partial(
        pl.kernel,
        out_type=jax.ShapeDtypeStruct((c * n,), jnp.float32),
        mesh=_vmesh(),
        compiler_params=_sc_params(),
        scratch_types=[
            pltpu.VMEM((ecd,), jnp.int32),
            pltpu.VMEM((ecd,), jnp.int32),
            pltpu.VMEM((rpw * ecd,), jnp.float32),
            pltpu.VMEM((rpw * ecd,), jnp.float32),
            pltpu.VMEM((rpw * n,), jnp.float32),
            pltpu.SemaphoreType.DMA,
        ],
    )
    def k(ht_hbm, dst_hbm, out_hbm, db0, db1, hb0, hb1, acc, sd):
        w = lax.axis_index("s") * NC + lax.axis_index("c")
        rbase = w * rpw

        @pl.loop(0, rpw * n, step=LANES)
        def _(i):
            acc[pl.ds(i, LANES)] = jnp.zeros((LANES,), jnp.float32)

        def fire(off, db, hb):
            pltpu.async_copy(dst_hbm.at[pl.ds(off, ecd)], db, sd)
            for r in range(rpw):
                pltpu.async_copy(
                    ht_hbm.at[pl.ds((rbase + r) * e + off, ecd)],
                    hb.at[pl.ds(r * ecd, ecd)], sd)

        def drain(db, hb):
            pltpu.make_async_copy(dst_hbm.at[pl.ds(0, ecd)], db, sd).wait()
            for r in range(rpw):
                pltpu.make_async_copy(
                    ht_hbm.at[pl.ds(r * ecd, ecd)],
                    hb.at[pl.ds(r * ecd, ecd)], sd).wait()

        def process(db, hb):
            @pl.loop(0, ecd, step=LANES)
            def _(i):
                idx = db[pl.ds(i, LANES)]
                saved = []
                for r in range(rpw):
                    vals = hb[pl.ds(r * ecd + i, LANES)]
                    idxr = idx + jnp.full((LANES,), r * n, jnp.int32)
                    cur = plsc.load_gather(acc, [idxr])
                    new = jnp.maximum(cur, vals)
                    plsc.store_scatter(acc, [idxr], new)
                    saved.append((idxr, new))
                bad = None
                for idxr, new in saved:
                    chk = plsc.load_gather(acc, [idxr])
                    b = chk < new
                    bad = b if bad is None else (bad | b)
                nbad = plsc.all_reduce_population_count(bad)[0]

                @pl.when(nbad > 0)
                def _():
                    for idxr, new in saved:
                        _repair(acc, idxr, new)

        fire(0, db0, hb0)

        @pl.loop(0, e, step=2 * ecd)
        def _(off):
            drain(db0, hb0)
            fire(off + ecd, db1, hb1)
            process(db0, hb0)
            drain(db1, hb1)

            @pl.when(off + 2 * ecd < e)
            def _():
                fire(off + 2 * ecd, db0, hb0)

            process(db1, hb1)

        pltpu.sync_copy(acc, out_hbm.at[pl.ds(rbase * n, rpw * n)])

    return k(ht1d, dst)


def _pq2_body(c, ht_ref, w_ref, b_ref, p2_ref, q2_ref):
    h = ht_ref[...]
    wa = w_ref[:c, :]
    wb = w_ref[c:, :]
    q2 = lax.dot_general(h, wb, (((0,), (0,)), ((), ())),
                         preferred_element_type=jnp.float32)
    p2 = lax.dot_general(h, wa, (((0,), (0,)), ((), ())),
                         preferred_element_type=jnp.float32) - q2 + b_ref[...]
    p2_ref[...] = p2
    q2_ref[...] = q2


def _stage_e(ht, w, b, n, c, out):
    return pl.pallas_call(
        functools.partial(_pq2_body, c),
        out_shape=(jax.ShapeDtypeStruct((n, out), jnp.float32),
                   jax.ShapeDtypeStruct((n, out), jnp.float32)),
    )(ht, w, b.reshape(1, out))


def _stage_f(p2f, q2f, dst, src, wv, n, e):
    epw = e // NW
    ecf = 2000
    assert epw % ecf == 0
    n2 = 2 * n

    @functools.partial(
        pl.kernel,
        out_type=jax.ShapeDtypeStruct((NW * n2,), jnp.float32),
        mesh=_vmesh(),
        compiler_params=_sc_params(),
        scratch_types=[
            pltpu.VMEM((n2,), jnp.float32),
            pltpu.VMEM((n2,), jnp.float32),
            pltpu.VMEM((n2,), jnp.float32),
            pltpu.VMEM((ecf,), jnp.int32),
            pltpu.VMEM((ecf,), jnp.int32),
            pltpu.VMEM((6 * LANES,), jnp.float32),
        ],
    )
    def k(p2_hbm, q2_hbm, dst_hbm, src_hbm, wv_hbm, part_hbm,
          p2b, q2b, accf, db, sb, wb6):
        w = lax.axis_index("s") * NC + lax.axis_index("c")
        base = w * epw
        pltpu.sync_copy(p2_hbm, p2b)
        pltpu.sync_copy(q2_hbm, q2b)
        pltpu.sync_copy(wv_hbm, wb6)

        @pl.loop(0, n2, step=LANES)
        def _(i):
            accf[pl.ds(i, LANES)] = jnp.full((LANES,), _NEG_INF, jnp.float32)

        w00 = wb6[pl.ds(0, LANES)]
        w01 = wb6[pl.ds(16, LANES)]
        w10 = wb6[pl.ds(32, LANES)]
        w11 = wb6[pl.ds(48, LANES)]
        b0 = wb6[pl.ds(64, LANES)]
        b1 = wb6[pl.ds(80, LANES)]

        @pl.loop(0, epw, step=ecf)
        def _(off):
            pltpu.sync_copy(dst_hbm.at[pl.ds(base + off, ecf)], db)
            pltpu.sync_copy(src_hbm.at[pl.ds(base + off, ecf)], sb)

            @pl.loop(0, ecf, step=LANES)
            def _(i):
                d = db[pl.ds(i, LANES)]
                s = sb[pl.ds(i, LANES)]
                one = jnp.ones((LANES,), jnp.int32)
                d2 = d + d
                s2 = s + s
                p0 = plsc.load_gather(p2b, [d2])
                p1 = plsc.load_gather(p2b, [d2 + one])
                q0 = plsc.load_gather(q2b, [s2])
                q1 = plsc.load_gather(q2b, [s2 + one])
                g0 = jnp.maximum(p0 + q0, 0.0)
                g1 = jnp.maximum(p1 + q1, 0.0)
                o0 = g0 * w00 + g1 * w10 + b0
                o1 = g0 * w01 + g1 * w11 + b1
                _scatter_max1(accf, d2, o0)
                _scatter_max1(accf, d2 + one, o1)

        pltpu.sync_copy(accf, part_hbm.at[pl.ds(w * n2, n2)])

    return k(p2f, q2f, dst, src, wv)


def _combine_body(p_ref, o_ref):
    m = jnp.max(p_ref[...], axis=0, keepdims=True)
    o_ref[...] = jnp.where(m == _NEG_INF, 0.0, m)


def _stage_g(parts, n):
    n2 = 2 * n
    return pl.pallas_call(
        _combine_body,
        out_shape=jax.ShapeDtypeStruct((1, n2), jnp.float32),
    )(parts)


def kernel(x, edge_index, l1_W1, l1_b1, l1_W2, l1_b2,
           l2_W1, l2_b1, l2_W2, l2_b2):
    n, f = x.shape
    e = edge_index.shape[1]
    c = l1_W2.shape[0]
    out = l2_W2.shape[0]

    src = edge_index[0]
    dst = edge_index[1]

    p, q = _stage_a(x, l1_W1, l1_b1, n, f, c)
    pg, qg = _stage_b(p, q, dst, src, e, c)
    ht = _stage_c(pg, qg, l1_W2, l1_b2, e, c)
    h_t = _stage_d(ht.reshape(-1), dst, n, e, c).reshape(c, n)
    p2, q2 = _stage_e(h_t, l2_W1, l2_b1, n, c, out)

    wv = jnp.concatenate(
        [jnp.broadcast_to(l2_W2.reshape(4, 1), (4, LANES)),
         jnp.broadcast_to(l2_b2.reshape(2, 1), (2, LANES))],
        axis=0).reshape(-1)
    parts = _stage_f(p2.reshape(-1), q2.reshape(-1), dst, src, wv, n, e)
    res = _stage_g(parts.reshape(NW, 2 * n), n)
    return res.reshape(n, out)


# speedup vs baseline: 4.1584x; 1.3377x over previous
import dataclasses
import functools

import jax
import jax.numpy as jnp
from jax import lax
from jax.experimental import pallas as pl
from jax.experimental.pallas import tpu as pltpu
from jax.experimental.pallas import tpu_sc as plsc

NC = 2
NS = 16
NW = NC * NS
LANES = 16

_NEG_INF = float("-inf")


def _vmesh():
    return plsc.VectorSubcoreMesh(core_axis_name="c", subcore_axis_name="s")


def _sc_params():
    cp = pltpu.CompilerParams()
    if "needs_layout_passes" in pltpu.CompilerParams.__dataclass_fields__:
        cp = dataclasses.replace(cp, needs_layout_passes=False)
    return cp


def _pq_body(f, x_ref, w1_ref, b1_ref, p_ref, q_ref):
    xv = x_ref[...]
    wa = w1_ref[:f, :]
    wb = w1_ref[f:, :]
    qv = jnp.dot(xv, wb, preferred_element_type=jnp.float32)
    p_ref[...] = jnp.dot(xv, wa, preferred_element_type=jnp.float32) - qv \
        + b1_ref[...]
    q_ref[...] = qv


def _stage_a(x, w1, b1, n, f, c):
    return pl.pallas_call(
        functools.partial(_pq_body, f),
        out_shape=(jax.ShapeDtypeStruct((n, c), jnp.float32),
                   jax.ShapeDtypeStruct((n, c), jnp.float32)),
    )(x, w1, b1.reshape(1, c))


def _stage_b(p, q, dst, src, e, c):
    epw = e // NW
    ecb = 200
    assert epw % ecb == 0

    sub = 40
    nsub = ecb // sub
    nch = epw // ecb
    npair = nch // 2

    @functools.partial(
        pl.kernel,
        out_type=(jax.ShapeDtypeStruct((e, c), jnp.float32),
                  jax.ShapeDtypeStruct((e, c), jnp.float32)),
        mesh=_vmesh(),
        compiler_params=_sc_params(),
        scratch_types=[
            pltpu.VMEM((epw,), jnp.int32),
            pltpu.VMEM((epw,), jnp.int32),
            pltpu.VMEM((ecb, c), jnp.float32),
            pltpu.VMEM((ecb, c), jnp.float32),
            pltpu.VMEM((ecb, c), jnp.float32),
            pltpu.VMEM((ecb, c), jnp.float32),
            pltpu.SemaphoreType.DMA,
            pltpu.SemaphoreType.DMA,
            pltpu.SemaphoreType.DMA,
        ],
    )
    def k(p_hbm, q_hbm, dst_hbm, src_hbm, pg_hbm, qg_hbm,
          dia, sia, pb0, qb0, pb1, qb1, sg, sw0, sw1):
        w = lax.axis_index("s") * NC + lax.axis_index("c")
        base = w * epw
        pltpu.sync_copy(dst_hbm.at[pl.ds(base, epw)], dia)
        pltpu.sync_copy(src_hbm.at[pl.ds(base, epw)], sia)

        def gathers(off, pb, qb):
            for j in range(nsub):
                pltpu.async_copy(
                    p_hbm.at[dia.at[pl.ds(off + j * sub, sub)]],
                    pb.at[pl.ds(j * sub, sub)], sg)
                pltpu.async_copy(
                    q_hbm.at[sia.at[pl.ds(off + j * sub, sub)]],
                    qb.at[pl.ds(j * sub, sub)], sg)

        def drain_gathers(pb, qb):
            for j in range(nsub):
                pltpu.make_async_copy(
                    p_hbm.at[dia.at[pl.ds(j * sub, sub)]],
                    pb.at[pl.ds(j * sub, sub)], sg).wait()
                pltpu.make_async_copy(
                    q_hbm.at[sia.at[pl.ds(j * sub, sub)]],
                    qb.at[pl.ds(j * sub, sub)], sg).wait()

        def write(off, pb, qb, sw):
            pltpu.async_copy(pb, pg_hbm.at[pl.ds(base + off, ecb)], sw)
            pltpu.async_copy(qb, qg_hbm.at[pl.ds(base + off, ecb)], sw)

        def wait_write(off, pb, qb, sw):
            pltpu.make_async_copy(pb, pg_hbm.at[pl.ds(base + off, ecb)],
                                  sw).wait()
            pltpu.make_async_copy(qb, qg_hbm.at[pl.ds(base + off, ecb)],
                                  sw).wait()

        gathers(0, pb0, qb0)

        @pl.loop(0, npair * 2 * ecb, step=2 * ecb)
        def _(off):
            drain_gathers(pb0, qb0)

            @pl.when(off > 0)
            def _():
                wait_write(off - ecb, pb1, qb1, sw1)

            gathers(off + ecb, pb1, qb1)
            write(off, pb0, qb0, sw0)
            drain_gathers(pb1, qb1)
            wait_write(off, pb0, qb0, sw0)

            @pl.when(off + 2 * ecb < epw)
            def _():
                gathers(off + 2 * ecb, pb0, qb0)

            write(off + ecb, pb1, qb1, sw1)

        if nch % 2 == 1:
            off_t = (nch - 1) * ecb
            drain_gathers(pb0, qb0)
            write(off_t, pb0, qb0, sw0)
            wait_write(off_t, pb0, qb0, sw0)
        wait_write((2 * npair - 1) * ecb, pb1, qb1, sw1)

    return k(p, q, dst, src)


def _ht_body(f, xi_ref, xj_ref, w1_ref, b1_ref, w2_ref, b2_ref, ht_ref):
    xi = xi_ref[...]
    xj = xj_ref[...]
    wa = w1_ref[:f, :]
    wb = w1_ref[f:, :]
    h1 = jnp.maximum(
        jnp.dot(xi, wa, preferred_element_type=jnp.float32)
        + jnp.dot(xj - xi, wb, preferred_element_type=jnp.float32)
        + b1_ref[...], 0.0)
    ht_ref[...] = lax.dot_general(
        w2_ref[...], h1, (((0,), (1,)), ((), ())),
        preferred_element_type=jnp.float32) + b2_ref[...]


def _stage_c(xi, xj, w1, b1, w2, b2, e, c, f):
    be = 1280
    assert e % be == 0
    return pl.pallas_call(
        functools.partial(_ht_body, f),
        grid=(e // be,),
        in_specs=[
            pl.BlockSpec((be, f), lambda i: (i, 0)),
            pl.BlockSpec((be, f), lambda i: (i, 0)),
            pl.BlockSpec((2 * f, c), lambda i: (0, 0)),
            pl.BlockSpec((1, c), lambda i: (0, 0)),
            pl.BlockSpec((c, c), lambda i: (0, 0)),
            pl.BlockSpec((c, 1), lambda i: (0, 0)),
        ],
        out_specs=pl.BlockSpec((c, be), lambda i: (0, i)),
        out_shape=jax.ShapeDtypeStruct((c, e), jnp.float32),
    )(xi, xj, w1, b1.reshape(1, c), w2, b2.reshape(c, 1))


def _repair(acc, idx, want0):
    def cond(cs):
        return cs[0] > 0

    def body(cs):
        _, want = cs
        c2 = plsc.load_gather(acc, [idx])
        plsc.store_scatter(acc, [idx], jnp.maximum(c2, want), mask=c2 < want)
        c3 = plsc.load_gather(acc, [idx])
        nb = plsc.all_reduce_population_count(c3 < want)[0]
        return nb, want

    lax.while_loop(cond, body, (jnp.int32(1), want0))


def _scatter_max1(acc, idx, vals):
    cur = plsc.load_gather(acc, [idx])
    new = jnp.maximum(cur, vals)
    plsc.store_scatter(acc, [idx], new)
    chk = plsc.load_gather(acc, [idx])
    nbad = plsc.all_reduce_population_count(chk < new)[0]

    @pl.when(nbad > 0)
    def _():
        _repair(acc, idx, new)


def _stage_d(ht1d, dst, n, e, c, init1d=None):
    rpw = c // NW
    ecd = 3200
    bv = 10
    assert (e // ecd) % 2 == 0 and ecd % (bv * LANES) == 0

    init_args = () if init1d is None else (init1d,)

    @functools.partial(
        pl.kernel,
        out_type=jax.ShapeDtypeStruct((c * n,), jnp.float32),
        mesh=_vmesh(),
        compiler_params=_sc_params(),
        scratch_types=[
            pltpu.VMEM((ecd,), jnp.int32),
            pltpu.VMEM((ecd,), jnp.int32),
            pltpu.VMEM((rpw * ecd,), jnp.float32),
            pltpu.VMEM((rpw * ecd,), jnp.float32),
            pltpu.VMEM((rpw * n,), jnp.float32),
            pltpu.SemaphoreType.DMA,
        ],
    )
    def k(ht_hbm, dst_hbm, *rest):
        if init1d is None:
            (out_hbm, db0, db1, hb0, hb1, acc, sd) = rest
            init_hbm = None
        else:
            (init_hbm, out_hbm, db0, db1, hb0, hb1, acc, sd) = rest
        w = lax.axis_index("s") * NC + lax.axis_index("c")
        rbase = w * rpw

        if init1d is None:
            @pl.loop(0, rpw * n, step=LANES)
            def _(i):
                acc[pl.ds(i, LANES)] = jnp.zeros((LANES,), jnp.float32)
        else:
            pltpu.sync_copy(init_hbm.at[pl.ds(rbase * n, rpw * n)], acc)

        def fire(off, db, hb):
            pltpu.async_copy(dst_hbm.at[pl.ds(off, ecd)], db, sd)
            for r in range(rpw):
                pltpu.async_copy(
                    ht_hbm.at[pl.ds((rbase + r) * e + off, ecd)],
                    hb.at[pl.ds(r * ecd, ecd)], sd)

        def drain(db, hb):
            pltpu.make_async_copy(dst_hbm.at[pl.ds(0, ecd)], db, sd).wait()
            for r in range(rpw):
                pltpu.make_async_copy(
                    ht_hbm.at[pl.ds(r * ecd, ecd)],
                    hb.at[pl.ds(r * ecd, ecd)], sd).wait()

        def process(db, hb):
            @pl.loop(0, ecd, step=bv * LANES)
            def _(i):
                bad = None
                for v in range(bv):
                    idx = db[pl.ds(i + v * LANES, LANES)]
                    for r in range(rpw):
                        vals = hb[pl.ds(r * ecd + i + v * LANES, LANES)]
                        idxr = idx + jnp.full((LANES,), r * n, jnp.int32)
                        cur = plsc.load_gather(acc, [idxr])
                        new = jnp.maximum(cur, vals)
                        plsc.store_scatter(acc, [idxr], new)
                        chk = plsc.load_gather(acc, [idxr])
                        b = chk < new
                        bad = b if bad is None else (bad | b)
                nbad = plsc.all_reduce_population_count(bad)[0]

                @pl.when(nbad > 0)
                def _():
                    for v in range(bv):
                        idx = db[pl.ds(i + v * LANES, LANES)]
                        for r in range(rpw):
                            vals = hb[pl.ds(r * ecd + i + v * LANES, LANES)]
                            idxr = idx + jnp.full((LANES,), r * n, jnp.int32)
                            _repair(acc, idxr, vals)

        fire(0, db0, hb0)

        @pl.loop(0, e, step=2 * ecd)
        def _(off):
            drain(db0, hb0)
            fire(off + ecd, db1, hb1)
            process(db0, hb0)
            drain(db1, hb1)

            @pl.when(off + 2 * ecd < e)
            def _():
                fire(off + 2 * ecd, db0, hb0)

            process(db1, hb1)

        pltpu.sync_copy(acc, out_hbm.at[pl.ds(rbase * n, rpw * n)])

    return k(ht1d, dst, *init_args)


def _pq2_body(c, ht_ref, w_ref, b_ref, p2_ref, q2_ref):
    h = ht_ref[...]
    wa = w_ref[:c, :]
    wb = w_ref[c:, :]
    q2 = lax.dot_general(h, wb, (((0,), (0,)), ((), ())),
                         preferred_element_type=jnp.float32)
    p2 = lax.dot_general(h, wa, (((0,), (0,)), ((), ())),
                         preferred_element_type=jnp.float32) - q2 + b_ref[...]
    p2_ref[...] = p2
    q2_ref[...] = q2


def _stage_e(ht, w, b, n, c, out):
    return pl.pallas_call(
        functools.partial(_pq2_body, c),
        out_shape=(jax.ShapeDtypeStruct((n, out), jnp.float32),
                   jax.ShapeDtypeStruct((n, out), jnp.float32)),
    )(ht, w, b.reshape(1, out))


def _stage_f(p2f, q2f, dst, src, wv, n, e):
    epw = e // NW
    ecf = 2000
    assert epw % ecf == 0
    n2 = 2 * n

    @functools.partial(
        pl.kernel,
        out_type=jax.ShapeDtypeStruct((NW * n2,), jnp.float32),
        mesh=_vmesh(),
        compiler_params=_sc_params(),
        scratch_types=[
            pltpu.VMEM((n2,), jnp.float32),
            pltpu.VMEM((n2,), jnp.float32),
            pltpu.VMEM((n2,), jnp.float32),
            pltpu.VMEM((ecf,), jnp.int32),
            pltpu.VMEM((ecf,), jnp.int32),
            pltpu.VMEM((6 * LANES,), jnp.float32),
        ],
    )
    def k(p2_hbm, q2_hbm, dst_hbm, src_hbm, wv_hbm, part_hbm,
          p2b, q2b, accf, db, sb, wb6):
        w = lax.axis_index("s") * NC + lax.axis_index("c")
        base = w * epw
        pltpu.sync_copy(p2_hbm, p2b)
        pltpu.sync_copy(q2_hbm, q2b)
        pltpu.sync_copy(wv_hbm, wb6)

        @pl.loop(0, n2, step=LANES)
        def _(i):
            accf[pl.ds(i, LANES)] = jnp.full((LANES,), _NEG_INF, jnp.float32)

        w00 = wb6[pl.ds(0, LANES)]
        w01 = wb6[pl.ds(16, LANES)]
        w10 = wb6[pl.ds(32, LANES)]
        w11 = wb6[pl.ds(48, LANES)]
        b0 = wb6[pl.ds(64, LANES)]
        b1 = wb6[pl.ds(80, LANES)]

        @pl.loop(0, epw, step=ecf)
        def _(off):
            pltpu.sync_copy(dst_hbm.at[pl.ds(base + off, ecf)], db)
            pltpu.sync_copy(src_hbm.at[pl.ds(base + off, ecf)], sb)

            @pl.loop(0, ecf, step=LANES)
            def _(i):
                d = db[pl.ds(i, LANES)]
                s = sb[pl.ds(i, LANES)]
                one = jnp.ones((LANES,), jnp.int32)
                d2 = d + d
                s2 = s + s
                p0 = plsc.load_gather(p2b, [d2])
                p1 = plsc.load_gather(p2b, [d2 + one])
                q0 = plsc.load_gather(q2b, [s2])
                q1 = plsc.load_gather(q2b, [s2 + one])
                g0 = jnp.maximum(p0 + q0, 0.0)
                g1 = jnp.maximum(p1 + q1, 0.0)
                o0 = g0 * w00 + g1 * w10 + b0
                o1 = g0 * w01 + g1 * w11 + b1
                _scatter_max1(accf, d2, o0)
                _scatter_max1(accf, d2 + one, o1)

        pltpu.sync_copy(accf, part_hbm.at[pl.ds(w * n2, n2)])

    return k(p2f, q2f, dst, src, wv)


def _combine_body(p_ref, o_ref):
    m = jnp.max(p_ref[...], axis=0, keepdims=True)
    o_ref[...] = jnp.where(m == _NEG_INF, 0.0, m)


def _stage_g(parts, n):
    n2 = 2 * n
    return pl.pallas_call(
        _combine_body,
        out_shape=jax.ShapeDtypeStruct((1, n2), jnp.float32),
    )(parts)


def kernel(x, edge_index, l1_W1, l1_b1, l1_W2, l1_b2,
           l2_W1, l2_b1, l2_W2, l2_b2):
    n, f = x.shape
    e = edge_index.shape[1]
    c = l1_W2.shape[0]
    out = l2_W2.shape[0]

    src = edge_index[0]
    dst = edge_index[1]

    e2 = e // 2
    dst0, dst1 = dst[:e2], dst[e2:]
    src0, src1 = src[:e2], src[e2:]
    xi0, xj0 = _stage_b(x, x, dst0, src0, e2, f)
    xi1, xj1 = _stage_b(x, x, dst1, src1, e2, f)
    ht0 = _stage_c(xi0, xj0, l1_W1, l1_b1, l1_W2, l1_b2, e2, c, f)
    ht1 = _stage_c(xi1, xj1, l1_W1, l1_b1, l1_W2, l1_b2, e2, c, f)
    hpart = _stage_d(ht0.reshape(-1), dst0, n, e2, c)
    h_t = _stage_d(ht1.reshape(-1), dst1, n, e2, c, init1d=hpart).reshape(c, n)
    p2, q2 = _stage_e(h_t, l2_W1, l2_b1, n, c, out)

    wv = jnp.concatenate(
        [jnp.broadcast_to(l2_W2.reshape(4, 1), (4, LANES)),
         jnp.broadcast_to(l2_b2.reshape(2, 1), (2, LANES))],
        axis=0).reshape(-1)
    parts = _stage_f(p2.reshape(-1), q2.reshape(-1), dst, src, wv, n, e)
    res = _stage_g(parts.reshape(NW, 2 * n), n)
    return res.reshape(n, out)
